# baseline (speedup 1.0000x reference)
import functools
import jax
import jax.numpy as jnp
from jax import lax
from jax.experimental import pallas as pl
from jax.experimental.pallas import tpu as pltpu
from jax.experimental.pallas import tpu_sc as plsc

NB = 16384
NH = 50
D = 64
NW = 32
BW = NB // NW
HB = 256
L = 16
NST = NH * 2

_mesh = plsc.VectorSubcoreMesh(core_axis_name="c", subcore_axis_name="s")


@functools.partial(
    pl.kernel, mesh=_mesh,
    out_type=jax.ShapeDtypeStruct((NH, D, NB), jnp.float32),
    scratch_types=[
        pltpu.VMEM((HB,), jnp.int32),
        pltpu.VMEM((HB,), jnp.int32),
        pltpu.VMEM((HB,), jnp.int32),
        pltpu.VMEM((HB,), jnp.int32),
        pltpu.VMEM((HB,), jnp.int32),
        pltpu.VMEM((HB,), jnp.int32),
        pltpu.VMEM((HB, 128), jnp.float32),
        pltpu.VMEM((HB, 128), jnp.float32),
        pltpu.VMEM((D, HB), jnp.float32),
        pltpu.VMEM((D, HB), jnp.float32),
        pltpu.SemaphoreType.DMA,
        pltpu.SemaphoreType.DMA,
        pltpu.SemaphoreType.DMA,
        pltpu.SemaphoreType.DMA,
        pltpu.SemaphoreType.DMA,
        pltpu.SemaphoreType.DMA,
    ],
    compiler_params=pltpu.CompilerParams(use_tc_tiling_on_sc=True,
                                         needs_layout_passes=False),
)
def _emb2(w128, idx_t, out_hbm,
          idx0, idx1, row0, row1, par0, par1, g0, g1, t0, t1,
          isem0, isem1, gsem0, gsem1, ssem0, ssem1):
    idx_v = (idx0, idx1)
    row_v = (row0, row1)
    par_v = (par0, par1)
    g_v = (g0, g1)
    t_v = (t0, t1)
    isems = (isem0, isem1)
    gsems = (gsem0, gsem1)
    ssems = (ssem0, ssem1)

    wid = lax.axis_index("s") * 2 + lax.axis_index("c")
    b0 = wid * BW

    def idx_load(h, p, b):
        pltpu.async_copy(idx_t.at[h, pl.ds(b0 + p * HB, HB)],
                         idx_v[b], isems[b])

    def split_and_gather(b):
        pltpu.make_async_copy(idx_t.at[0, pl.ds(0, HB)], idx_v[b],
                              isems[b]).wait()
        for k in range(HB // L):
            v = idx_v[b][pl.ds(k * L, L)]
            row_v[b][pl.ds(k * L, L)] = v >> 1
            par_v[b][pl.ds(k * L, L)] = (v & 1) << 6
        pltpu.async_copy(w128.at[row_v[b]], g_v[b], gsems[b])

    def extract(b):
        def xb(k, c):
            rows16 = jax.lax.broadcasted_iota(jnp.int32, (L,), 0) + k * L
            base = par_v[b][pl.ds(k * L, L)]
            for d0 in range(0, D, 8):
                vals = [plsc.load_gather(g_v[b], [rows16, base + d])
                        for d in range(d0, d0 + 8)]
                for i in range(8):
                    t_v[b][d0 + i, pl.ds(k * L, L)] = vals[i]
            return c
        lax.fori_loop(0, HB // L, xb, 0)

    idx_load(0, 0, 0)
    idx_load(0, 1, 1)
    split_and_gather(0)
    idx_load(1, 0, 0)

    def body(i, carry):
        @pl.when(i >= 1)
        def _():
            pltpu.make_async_copy(t_v[0], out_hbm.at[0, :, pl.ds(b0, HB)],
                                  ssems[0]).wait()
        split_and_gather(1)
        @pl.when(i + 1 < NH)
        def _():
            idx_load(i + 1, 1, 1)
        pltpu.make_async_copy(w128.at[row_v[0]], g_v[0], gsems[0]).wait()
        extract(0)
        pltpu.async_copy(t_v[0], out_hbm.at[i, :, pl.ds(b0, HB)], ssems[0])

        @pl.when(i >= 1)
        def _():
            pltpu.make_async_copy(t_v[1], out_hbm.at[0, :, pl.ds(b0, HB)],
                                  ssems[1]).wait()
        @pl.when(i + 1 < NH)
        def _():
            split_and_gather(0)
        @pl.when(i + 2 < NH)
        def _():
            idx_load(i + 2, 0, 0)
        pltpu.make_async_copy(w128.at[row_v[1]], g_v[1], gsems[1]).wait()
        extract(1)
        pltpu.async_copy(t_v[1], out_hbm.at[i, :, pl.ds(b0 + HB, HB)],
                         ssems[1])
        return carry

    lax.fori_loop(0, NH, body, 0)

    for b in range(2):
        pltpu.make_async_copy(t_v[b], out_hbm.at[0, :, pl.ds(b0, HB)],
                              ssems[b]).wait()


VOCAB = 1000000
NFB = VOCAB // 128


@functools.partial(
    pl.kernel, mesh=_mesh,
    out_type=jax.ShapeDtypeStruct((VOCAB // 2, 128), jnp.float32),
    scratch_types=[
        pltpu.VMEM((D, 128), jnp.float32),
        pltpu.VMEM((D, 128), jnp.float32),
        pltpu.VMEM((D, 128), jnp.float32),
        pltpu.VMEM((D, 128), jnp.float32),
        pltpu.SemaphoreType.DMA,
        pltpu.SemaphoreType.DMA,
        pltpu.SemaphoreType.DMA,
        pltpu.SemaphoreType.DMA,
    ],
    compiler_params=pltpu.CompilerParams(use_tc_tiling_on_sc=True,
                                         needs_layout_passes=False),
)
def _pack_rows(wt, tail32, w128_out, in0, in1, o0, o1,
               li0, li1, lo0, lo1):
    in_v = (in0, in1)
    o_v = (o0, o1)
    isems = (li0, li1)
    osems = (lo0, lo1)

    wid = lax.axis_index("s") * 2 + lax.axis_index("c")

    def load_blk(c, b):
        pltpu.async_copy(wt.at[:, pl.ds(c * 128, 128)], in_v[b], isems[b])

    def transpose_blk(b, nrow):
        def row(jj, carry):
            for half in range(2):
                col = 2 * jj + half
                for d0 in range(0, D, L):
                    rows16 = jax.lax.broadcasted_iota(
                        jnp.int32, (L,), 0) + d0
                    vals = plsc.load_gather(
                        in_v[b], [rows16, jnp.full((L,), 0, jnp.int32) + col])
                    o_v[b][jj, pl.ds(half * D + d0, L)] = vals
            return carry
        lax.fori_loop(0, nrow, row, 0)

    def store_blk(c, b, nrow):
        pltpu.async_copy(o_v[b].at[pl.ds(0, nrow)],
                         w128_out.at[pl.ds(c * D, nrow)], osems[b])

    nit = NFB // NW + 1
    load_blk(wid, 0)

    def body(i, carry):
        for b in range(2):
            ii = 2 * i + b
            c = wid + ii * NW

            @pl.when(c < NFB)
            def _():
                pltpu.make_async_copy(wt.at[:, pl.ds(0, 128)], in_v[b],
                                      isems[b]).wait()
                cn = wid + (ii + 1) * NW

                @pl.when(cn < NFB)
                def _():
                    load_blk(cn, 1 - b)

                @pl.when(ii >= 2)
                def _():
                    pltpu.make_async_copy(
                        o_v[b].at[pl.ds(0, D)],
                        w128_out.at[pl.ds(0, D)], osems[b]).wait()
                transpose_blk(b, D)
                store_blk(c, b, D)
        return carry

    lax.fori_loop(0, (nit + 1) // 2, body, 0)

    for b in range(2):
        pltpu.make_async_copy(o_v[b].at[pl.ds(0, D)],
                              w128_out.at[pl.ds(0, D)], osems[b]).wait()

    @pl.when(wid == 4)
    def _():
        pltpu.async_copy(tail32, o0.at[pl.ds(0, 32)], li0).wait()
        pltpu.async_copy(o0.at[pl.ds(0, 32)],
                         w128_out.at[pl.ds(NFB * D, 32)], lo0).wait()


def kernel(input_, weight):
    tail32 = weight[NFB * 128:].reshape(32, 128)
    w128 = _pack_rows(weight.T, tail32)
    idx_t = input_.T
    out = _emb2(w128, idx_t)
    return out.transpose(2, 0, 1)


# speedup vs baseline: 1.2176x; 1.2176x over previous
import functools
import jax
import jax.numpy as jnp
from jax import lax
from jax.experimental import pallas as pl
from jax.experimental.pallas import tpu as pltpu
from jax.experimental.pallas import tpu_sc as plsc

NB = 16384
NH = 50
D = 64
NW = 32
BW = NB // NW
HB = 256
L = 16
NST = NH * 2

_mesh = plsc.VectorSubcoreMesh(core_axis_name="c", subcore_axis_name="s")


@functools.partial(
    pl.kernel, mesh=_mesh,
    out_type=jax.ShapeDtypeStruct((NH, D, NB), jnp.float32),
    scratch_types=[
        pltpu.VMEM((HB,), jnp.int32),
        pltpu.VMEM((HB,), jnp.int32),
        pltpu.VMEM((HB,), jnp.int32),
        pltpu.VMEM((HB,), jnp.int32),
        pltpu.VMEM((HB,), jnp.int32),
        pltpu.VMEM((HB,), jnp.int32),
        pltpu.VMEM((HB, 128), jnp.float32),
        pltpu.VMEM((HB, 128), jnp.float32),
        pltpu.VMEM((D, HB), jnp.float32),
        pltpu.VMEM((D, HB), jnp.float32),
        pltpu.SemaphoreType.DMA,
        pltpu.SemaphoreType.DMA,
        pltpu.SemaphoreType.DMA,
        pltpu.SemaphoreType.DMA,
        pltpu.SemaphoreType.DMA,
        pltpu.SemaphoreType.DMA,
    ],
    compiler_params=pltpu.CompilerParams(use_tc_tiling_on_sc=True,
                                         needs_layout_passes=False),
)
def _emb2(w128, idx_t, out_hbm,
          idx0, idx1, row0, row1, par0, par1, g0, g1, t0, t1,
          isem0, isem1, gsem0, gsem1, ssem0, ssem1):
    idx_v = (idx0, idx1)
    row_v = (row0, row1)
    par_v = (par0, par1)
    g_v = (g0, g1)
    t_v = (t0, t1)
    isems = (isem0, isem1)
    gsems = (gsem0, gsem1)
    ssems = (ssem0, ssem1)

    wid = lax.axis_index("s") * 2 + lax.axis_index("c")
    b0 = wid * BW

    def idx_load(h, p, b):
        pltpu.async_copy(idx_t.at[h, pl.ds(b0 + p * HB, HB)],
                         idx_v[b], isems[b])

    def split_and_gather(b):
        pltpu.make_async_copy(idx_t.at[0, pl.ds(0, HB)], idx_v[b],
                              isems[b]).wait()
        for k in range(HB // L):
            v = idx_v[b][pl.ds(k * L, L)]
            row_v[b][pl.ds(k * L, L)] = v >> 1
            par_v[b][pl.ds(k * L, L)] = (v & 1) << 6
        pltpu.async_copy(w128.at[row_v[b]], g_v[b], gsems[b])

    def extract(b):
        def xb(k, c):
            rows16 = jax.lax.broadcasted_iota(jnp.int32, (L,), 0) + k * L
            base = par_v[b][pl.ds(k * L, L)]
            for d0 in range(0, D, 8):
                vals = [plsc.load_gather(g_v[b], [rows16, base + d])
                        for d in range(d0, d0 + 8)]
                for i in range(8):
                    t_v[b][d0 + i, pl.ds(k * L, L)] = vals[i]
            return c
        lax.fori_loop(0, HB // L, xb, 0)

    idx_load(0, 0, 0)
    idx_load(0, 1, 1)
    split_and_gather(0)
    idx_load(1, 0, 0)

    def body(i, carry):
        @pl.when(i >= 1)
        def _():
            pltpu.make_async_copy(t_v[0], out_hbm.at[0, :, pl.ds(b0, HB)],
                                  ssems[0]).wait()
        split_and_gather(1)
        @pl.when(i + 1 < NH)
        def _():
            idx_load(i + 1, 1, 1)
        pltpu.make_async_copy(w128.at[row_v[0]], g_v[0], gsems[0]).wait()
        extract(0)
        pltpu.async_copy(t_v[0], out_hbm.at[i, :, pl.ds(b0, HB)], ssems[0])

        @pl.when(i >= 1)
        def _():
            pltpu.make_async_copy(t_v[1], out_hbm.at[0, :, pl.ds(b0, HB)],
                                  ssems[1]).wait()
        @pl.when(i + 1 < NH)
        def _():
            split_and_gather(0)
        @pl.when(i + 2 < NH)
        def _():
            idx_load(i + 2, 0, 0)
        pltpu.make_async_copy(w128.at[row_v[1]], g_v[1], gsems[1]).wait()
        extract(1)
        pltpu.async_copy(t_v[1], out_hbm.at[i, :, pl.ds(b0 + HB, HB)],
                         ssems[1])
        return carry

    lax.fori_loop(0, NH, body, 0)

    for b in range(2):
        pltpu.make_async_copy(t_v[b], out_hbm.at[0, :, pl.ds(b0, HB)],
                              ssems[b]).wait()


VOCAB = 1000000
NFB = VOCAB // 128


@functools.partial(
    pl.kernel, mesh=_mesh,
    out_type=jax.ShapeDtypeStruct((VOCAB // 2, 128), jnp.float32),
    scratch_types=[
        pltpu.VMEM((D, 128), jnp.float32),
        pltpu.VMEM((D, 128), jnp.float32),
        pltpu.VMEM((D, 128), jnp.float32),
        pltpu.VMEM((D, 128), jnp.float32),
        pltpu.SemaphoreType.DMA,
        pltpu.SemaphoreType.DMA,
        pltpu.SemaphoreType.DMA,
        pltpu.SemaphoreType.DMA,
    ],
    compiler_params=pltpu.CompilerParams(use_tc_tiling_on_sc=True,
                                         needs_layout_passes=False),
)
def _pack_rows(wt, tail32, w128_out, in0, in1, o0, o1,
               li0, li1, lo0, lo1):
    in_v = (in0, in1)
    o_v = (o0, o1)
    isems = (li0, li1)
    osems = (lo0, lo1)

    wid = lax.axis_index("s") * 2 + lax.axis_index("c")

    def load_blk(c, b):
        pltpu.async_copy(wt.at[:, pl.ds(c * 128, 128)], in_v[b], isems[b])

    riota = jax.lax.broadcasted_iota(jnp.int32, (L,), 0)
    rows16s = [riota + d0 for d0 in range(0, D, L)]
    zeros = riota - riota

    def transpose_blk(b, nrow):
        def row2(q, carry):
            vals = []
            for u in range(2):
                jj = 2 * q + u
                for half in range(2):
                    col = zeros + (2 * jj + half)
                    for r16 in rows16s:
                        vals.append(plsc.load_gather(in_v[b], [r16, col]))
            k = 0
            for u in range(2):
                jj = 2 * q + u
                for half in range(2):
                    for i in range(4):
                        o_v[b][jj, pl.ds(half * D + i * L, L)] = vals[k]
                        k += 1
            return carry
        lax.fori_loop(0, nrow // 2, row2, 0)

    def store_blk(c, b, nrow):
        pltpu.async_copy(o_v[b].at[pl.ds(0, nrow)],
                         w128_out.at[pl.ds(c * D, nrow)], osems[b])

    nit = NFB // NW + 1
    load_blk(wid, 0)

    def body(i, carry):
        for b in range(2):
            ii = 2 * i + b
            c = wid + ii * NW

            @pl.when(c < NFB)
            def _():
                pltpu.make_async_copy(wt.at[:, pl.ds(0, 128)], in_v[b],
                                      isems[b]).wait()
                cn = wid + (ii + 1) * NW

                @pl.when(cn < NFB)
                def _():
                    load_blk(cn, 1 - b)

                @pl.when(ii >= 2)
                def _():
                    pltpu.make_async_copy(
                        o_v[b].at[pl.ds(0, D)],
                        w128_out.at[pl.ds(0, D)], osems[b]).wait()
                transpose_blk(b, D)
                store_blk(c, b, D)
        return carry

    lax.fori_loop(0, (nit + 1) // 2, body, 0)

    for b in range(2):
        pltpu.make_async_copy(o_v[b].at[pl.ds(0, D)],
                              w128_out.at[pl.ds(0, D)], osems[b]).wait()

    @pl.when(wid == 4)
    def _():
        pltpu.async_copy(tail32, o0.at[pl.ds(0, 32)], li0).wait()
        pltpu.async_copy(o0.at[pl.ds(0, 32)],
                         w128_out.at[pl.ds(NFB * D, 32)], lo0).wait()


def kernel(input_, weight):
    tail32 = weight[NFB * 128:].reshape(32, 128)
    w128 = _pack_rows(weight.T, tail32)
    idx_t = input_.T
    out = _emb2(w128, idx_t)
    return out.transpose(2, 0, 1)


# speedup vs baseline: 1.5438x; 1.2678x over previous
import functools
import jax
import jax.numpy as jnp
from jax import lax
from jax.experimental import pallas as pl
from jax.experimental.pallas import tpu as pltpu
from jax.experimental.pallas import tpu_sc as plsc

NB = 16384
NH = 50
D = 64
NW = 32
BW = NB // NW
HB = 256
L = 16
NST = NH * 2

_mesh = plsc.VectorSubcoreMesh(core_axis_name="c", subcore_axis_name="s")


@functools.partial(
    pl.kernel, mesh=_mesh,
    out_type=jax.ShapeDtypeStruct((NH, D, NB), jnp.float32),
    scratch_types=[
        pltpu.VMEM((HB,), jnp.int32),
        pltpu.VMEM((HB,), jnp.int32),
        pltpu.VMEM((HB,), jnp.int32),
        pltpu.VMEM((HB,), jnp.int32),
        pltpu.VMEM((HB,), jnp.int32),
        pltpu.VMEM((HB,), jnp.int32),
        pltpu.VMEM((HB, 128), jnp.float32),
        pltpu.VMEM((HB, 128), jnp.float32),
        pltpu.VMEM((D, HB), jnp.float32),
        pltpu.VMEM((D, HB), jnp.float32),
        pltpu.SemaphoreType.DMA,
        pltpu.SemaphoreType.DMA,
        pltpu.SemaphoreType.DMA,
        pltpu.SemaphoreType.DMA,
        pltpu.SemaphoreType.DMA,
        pltpu.SemaphoreType.DMA,
    ],
    compiler_params=pltpu.CompilerParams(use_tc_tiling_on_sc=True,
                                         needs_layout_passes=False),
)
def _emb2(w128, idx_t, out_hbm,
          idx0, idx1, row0, row1, par0, par1, g0, g1, t0, t1,
          isem0, isem1, gsem0, gsem1, ssem0, ssem1):
    idx_v = (idx0, idx1)
    row_v = (row0, row1)
    par_v = (par0, par1)
    g_v = (g0, g1)
    t_v = (t0, t1)
    isems = (isem0, isem1)
    gsems = (gsem0, gsem1)
    ssems = (ssem0, ssem1)

    wid = lax.axis_index("s") * 2 + lax.axis_index("c")
    b0 = wid * BW

    def idx_load(h, p, b):
        pltpu.async_copy(idx_t.at[h, pl.ds(b0 + p * HB, HB)],
                         idx_v[b], isems[b])

    def split_and_gather(b):
        pltpu.make_async_copy(idx_t.at[0, pl.ds(0, HB)], idx_v[b],
                              isems[b]).wait()
        for k in range(HB // L):
            v = idx_v[b][pl.ds(k * L, L)]
            row_v[b][pl.ds(k * L, L)] = v >> 1
            par_v[b][pl.ds(k * L, L)] = (v & 1) << 6
        pltpu.async_copy(w128.at[row_v[b]], g_v[b], gsems[b])

    def extract(b):
        def xb(k, c):
            rows16 = jax.lax.broadcasted_iota(jnp.int32, (L,), 0) + k * L
            base = par_v[b][pl.ds(k * L, L)]
            for d0 in range(0, D, 8):
                vals = [plsc.load_gather(g_v[b], [rows16, base + d])
                        for d in range(d0, d0 + 8)]
                for i in range(8):
                    t_v[b][d0 + i, pl.ds(k * L, L)] = vals[i]
            return c
        lax.fori_loop(0, HB // L, xb, 0)

    idx_load(0, 0, 0)
    idx_load(0, 1, 1)
    split_and_gather(0)
    idx_load(1, 0, 0)

    def body(i, carry):
        @pl.when(i >= 1)
        def _():
            pltpu.make_async_copy(t_v[0], out_hbm.at[0, :, pl.ds(b0, HB)],
                                  ssems[0]).wait()
        split_and_gather(1)
        @pl.when(i + 1 < NH)
        def _():
            idx_load(i + 1, 1, 1)
        pltpu.make_async_copy(w128.at[row_v[0]], g_v[0], gsems[0]).wait()
        extract(0)
        pltpu.async_copy(t_v[0], out_hbm.at[i, :, pl.ds(b0, HB)], ssems[0])

        @pl.when(i >= 1)
        def _():
            pltpu.make_async_copy(t_v[1], out_hbm.at[0, :, pl.ds(b0, HB)],
                                  ssems[1]).wait()
        @pl.when(i + 1 < NH)
        def _():
            split_and_gather(0)
        @pl.when(i + 2 < NH)
        def _():
            idx_load(i + 2, 0, 0)
        pltpu.make_async_copy(w128.at[row_v[1]], g_v[1], gsems[1]).wait()
        extract(1)
        pltpu.async_copy(t_v[1], out_hbm.at[i, :, pl.ds(b0 + HB, HB)],
                         ssems[1])
        return carry

    lax.fori_loop(0, NH, body, 0)

    for b in range(2):
        pltpu.make_async_copy(t_v[b], out_hbm.at[0, :, pl.ds(b0, HB)],
                              ssems[b]).wait()


VOCAB = 1000000
VC = 8000
NPB = VOCAB // VC


def _pack_body(w_ref, out_ref):
    x3 = w_ref[...].reshape(VC // 2, 2, D)
    out_ref[...] = jnp.concatenate([x3[:, 0, :], x3[:, 1, :]], axis=-1)


_pack_tc = pl.pallas_call(
    _pack_body,
    grid=(NPB,),
    in_specs=[pl.BlockSpec((VC, D), lambda i: (i, 0))],
    out_specs=pl.BlockSpec((VC // 2, 128), lambda i: (i, 0)),
    out_shape=jax.ShapeDtypeStruct((VOCAB // 2, 128), jnp.float32),
)


@functools.partial(
    pl.kernel, mesh=_mesh,
    out_type=jax.ShapeDtypeStruct((VOCAB // 2, 128), jnp.float32),
    scratch_types=[
        pltpu.VMEM((D, 128), jnp.float32),
        pltpu.VMEM((D, 128), jnp.float32),
        pltpu.VMEM((D, 128), jnp.float32),
        pltpu.VMEM((D, 128), jnp.float32),
        pltpu.SemaphoreType.DMA,
        pltpu.SemaphoreType.DMA,
        pltpu.SemaphoreType.DMA,
        pltpu.SemaphoreType.DMA,
    ],
    compiler_params=pltpu.CompilerParams(use_tc_tiling_on_sc=True,
                                         needs_layout_passes=False),
)
def _pack_rows(wt, tail32, w128_out, in0, in1, o0, o1,
               li0, li1, lo0, lo1):
    in_v = (in0, in1)
    o_v = (o0, o1)
    isems = (li0, li1)
    osems = (lo0, lo1)

    wid = lax.axis_index("s") * 2 + lax.axis_index("c")

    def load_blk(c, b):
        pltpu.async_copy(wt.at[:, pl.ds(c * 128, 128)], in_v[b], isems[b])

    riota = jax.lax.broadcasted_iota(jnp.int32, (L,), 0)
    rows16s = [riota + d0 for d0 in range(0, D, L)]
    zeros = riota - riota

    def transpose_blk(b, nrow):
        def row2(q, carry):
            vals = []
            for u in range(2):
                jj = 2 * q + u
                for half in range(2):
                    col = zeros + (2 * jj + half)
                    for r16 in rows16s:
                        vals.append(plsc.load_gather(in_v[b], [r16, col]))
            k = 0
            for u in range(2):
                jj = 2 * q + u
                for half in range(2):
                    for i in range(4):
                        o_v[b][jj, pl.ds(half * D + i * L, L)] = vals[k]
                        k += 1
            return carry
        lax.fori_loop(0, nrow // 2, row2, 0)

    def store_blk(c, b, nrow):
        pltpu.async_copy(o_v[b].at[pl.ds(0, nrow)],
                         w128_out.at[pl.ds(c * D, nrow)], osems[b])

    nit = NFB // NW + 1
    load_blk(wid, 0)

    def body(i, carry):
        for b in range(2):
            ii = 2 * i + b
            c = wid + ii * NW

            @pl.when(c < NFB)
            def _():
                pltpu.make_async_copy(wt.at[:, pl.ds(0, 128)], in_v[b],
                                      isems[b]).wait()
                cn = wid + (ii + 1) * NW

                @pl.when(cn < NFB)
                def _():
                    load_blk(cn, 1 - b)

                @pl.when(ii >= 2)
                def _():
                    pltpu.make_async_copy(
                        o_v[b].at[pl.ds(0, D)],
                        w128_out.at[pl.ds(0, D)], osems[b]).wait()
                transpose_blk(b, D)
                store_blk(c, b, D)
        return carry

    lax.fori_loop(0, (nit + 1) // 2, body, 0)

    for b in range(2):
        pltpu.make_async_copy(o_v[b].at[pl.ds(0, D)],
                              w128_out.at[pl.ds(0, D)], osems[b]).wait()

    @pl.when(wid == 4)
    def _():
        pltpu.async_copy(tail32, o0.at[pl.ds(0, 32)], li0).wait()
        pltpu.async_copy(o0.at[pl.ds(0, 32)],
                         w128_out.at[pl.ds(NFB * D, 32)], lo0).wait()


def kernel(input_, weight):
    w128 = _pack_tc(weight)
    idx_t = input_.T
    out = _emb2(w128, idx_t)
    return out.transpose(2, 0, 1)


# speedup vs baseline: 1.6887x; 1.0939x over previous
import functools
import jax
import jax.numpy as jnp
from jax import lax
from jax.experimental import pallas as pl
from jax.experimental.pallas import tpu as pltpu
from jax.experimental.pallas import tpu_sc as plsc

NB = 16384
NH = 50
D = 64
NW = 32
BW = NB // NW
HB = 256
L = 16
NST = NH * 2

_mesh = plsc.VectorSubcoreMesh(core_axis_name="c", subcore_axis_name="s")


@functools.partial(
    pl.kernel, mesh=_mesh,
    out_type=jax.ShapeDtypeStruct((NH, D, NB), jnp.float32),
    scratch_types=[
        pltpu.VMEM((HB,), jnp.int32),
        pltpu.VMEM((HB,), jnp.int32),
        pltpu.VMEM((HB,), jnp.int32),
        pltpu.VMEM((HB,), jnp.int32),
        pltpu.VMEM((HB,), jnp.int32),
        pltpu.VMEM((HB,), jnp.int32),
        pltpu.VMEM((HB, 128), jnp.float32),
        pltpu.VMEM((HB, 128), jnp.float32),
        pltpu.VMEM((D, HB), jnp.float32),
        pltpu.VMEM((D, HB), jnp.float32),
        pltpu.SemaphoreType.DMA,
        pltpu.SemaphoreType.DMA,
        pltpu.SemaphoreType.DMA,
        pltpu.SemaphoreType.DMA,
        pltpu.SemaphoreType.DMA,
        pltpu.SemaphoreType.DMA,
    ],
    compiler_params=pltpu.CompilerParams(use_tc_tiling_on_sc=True,
                                         needs_layout_passes=False),
)
def _emb2(w128, idx_t, out_hbm,
          idx0, idx1, row0, row1, par0, par1, g0, g1, t0, t1,
          isem0, isem1, gsem0, gsem1, ssem0, ssem1):
    idx_v = (idx0, idx1)
    row_v = (row0, row1)
    par_v = (par0, par1)
    g_v = (g0, g1)
    t_v = (t0, t1)
    isems = (isem0, isem1)
    gsems = (gsem0, gsem1)
    ssems = (ssem0, ssem1)

    wid = lax.axis_index("s") * 2 + lax.axis_index("c")
    b0 = wid * BW

    def idx_load(h, p, b):
        pltpu.async_copy(idx_t.at[h, pl.ds(b0 + p * HB, HB)],
                         idx_v[b], isems[b])

    def split_and_gather(b):
        pltpu.make_async_copy(idx_t.at[0, pl.ds(0, HB)], idx_v[b],
                              isems[b]).wait()
        for k in range(HB // L):
            v = idx_v[b][pl.ds(k * L, L)]
            row_v[b][pl.ds(k * L, L)] = v >> 1
            par_v[b][pl.ds(k * L, L)] = (v & 1) << 6
        pltpu.async_copy(w128.at[row_v[b]], g_v[b], gsems[b])

    def extract(b):
        def xb(k, c):
            rows16 = jax.lax.broadcasted_iota(jnp.int32, (L,), 0) + k * L
            base = par_v[b][pl.ds(k * L, L)]
            for d0 in range(0, D, 16):
                vals = [plsc.load_gather(g_v[b], [rows16, base + d])
                        for d in range(d0, d0 + 16)]
                for i in range(16):
                    t_v[b][d0 + i, pl.ds(k * L, L)] = vals[i]
            return c
        lax.fori_loop(0, HB // L, xb, 0)

    idx_load(0, 0, 0)
    idx_load(0, 1, 1)
    split_and_gather(0)
    idx_load(1, 0, 0)

    def body(i, carry):
        @pl.when(i >= 1)
        def _():
            pltpu.make_async_copy(t_v[0], out_hbm.at[0, :, pl.ds(b0, HB)],
                                  ssems[0]).wait()
        split_and_gather(1)
        @pl.when(i + 1 < NH)
        def _():
            idx_load(i + 1, 1, 1)
        pltpu.make_async_copy(w128.at[row_v[0]], g_v[0], gsems[0]).wait()
        extract(0)
        pltpu.async_copy(t_v[0], out_hbm.at[i, :, pl.ds(b0, HB)], ssems[0])

        @pl.when(i >= 1)
        def _():
            pltpu.make_async_copy(t_v[1], out_hbm.at[0, :, pl.ds(b0, HB)],
                                  ssems[1]).wait()
        @pl.when(i + 1 < NH)
        def _():
            split_and_gather(0)
        @pl.when(i + 2 < NH)
        def _():
            idx_load(i + 2, 0, 0)
        pltpu.make_async_copy(w128.at[row_v[1]], g_v[1], gsems[1]).wait()
        extract(1)
        pltpu.async_copy(t_v[1], out_hbm.at[i, :, pl.ds(b0 + HB, HB)],
                         ssems[1])
        return carry

    lax.fori_loop(0, NH, body, 0)

    for b in range(2):
        pltpu.make_async_copy(t_v[b], out_hbm.at[0, :, pl.ds(b0, HB)],
                              ssems[b]).wait()


VOCAB = 1000000
VC = 8000
NPB = VOCAB // VC


def _pack_body(w_ref, out_ref):
    x3 = w_ref[...].reshape(VC // 2, 2, D)
    out_ref[...] = jnp.concatenate([x3[:, 0, :], x3[:, 1, :]], axis=-1)


_pack_tc = pl.pallas_call(
    _pack_body,
    grid=(NPB,),
    in_specs=[pl.BlockSpec((VC, D), lambda i: (i, 0))],
    out_specs=pl.BlockSpec((VC // 2, 128), lambda i: (i, 0)),
    out_shape=jax.ShapeDtypeStruct((VOCAB // 2, 128), jnp.float32),
)


@functools.partial(
    pl.kernel, mesh=_mesh,
    out_type=jax.ShapeDtypeStruct((VOCAB // 2, 128), jnp.float32),
    scratch_types=[
        pltpu.VMEM((D, 128), jnp.float32),
        pltpu.VMEM((D, 128), jnp.float32),
        pltpu.VMEM((D, 128), jnp.float32),
        pltpu.VMEM((D, 128), jnp.float32),
        pltpu.SemaphoreType.DMA,
        pltpu.SemaphoreType.DMA,
        pltpu.SemaphoreType.DMA,
        pltpu.SemaphoreType.DMA,
    ],
    compiler_params=pltpu.CompilerParams(use_tc_tiling_on_sc=True,
                                         needs_layout_passes=False),
)
def _pack_rows(wt, tail32, w128_out, in0, in1, o0, o1,
               li0, li1, lo0, lo1):
    in_v = (in0, in1)
    o_v = (o0, o1)
    isems = (li0, li1)
    osems = (lo0, lo1)

    wid = lax.axis_index("s") * 2 + lax.axis_index("c")

    def load_blk(c, b):
        pltpu.async_copy(wt.at[:, pl.ds(c * 128, 128)], in_v[b], isems[b])

    riota = jax.lax.broadcasted_iota(jnp.int32, (L,), 0)
    rows16s = [riota + d0 for d0 in range(0, D, L)]
    zeros = riota - riota

    def transpose_blk(b, nrow):
        def row2(q, carry):
            vals = []
            for u in range(2):
                jj = 2 * q + u
                for half in range(2):
                    col = zeros + (2 * jj + half)
                    for r16 in rows16s:
                        vals.append(plsc.load_gather(in_v[b], [r16, col]))
            k = 0
            for u in range(2):
                jj = 2 * q + u
                for half in range(2):
                    for i in range(4):
                        o_v[b][jj, pl.ds(half * D + i * L, L)] = vals[k]
                        k += 1
            return carry
        lax.fori_loop(0, nrow // 2, row2, 0)

    def store_blk(c, b, nrow):
        pltpu.async_copy(o_v[b].at[pl.ds(0, nrow)],
                         w128_out.at[pl.ds(c * D, nrow)], osems[b])

    nit = NFB // NW + 1
    load_blk(wid, 0)

    def body(i, carry):
        for b in range(2):
            ii = 2 * i + b
            c = wid + ii * NW

            @pl.when(c < NFB)
            def _():
                pltpu.make_async_copy(wt.at[:, pl.ds(0, 128)], in_v[b],
                                      isems[b]).wait()
                cn = wid + (ii + 1) * NW

                @pl.when(cn < NFB)
                def _():
                    load_blk(cn, 1 - b)

                @pl.when(ii >= 2)
                def _():
                    pltpu.make_async_copy(
                        o_v[b].at[pl.ds(0, D)],
                        w128_out.at[pl.ds(0, D)], osems[b]).wait()
                transpose_blk(b, D)
                store_blk(c, b, D)
        return carry

    lax.fori_loop(0, (nit + 1) // 2, body, 0)

    for b in range(2):
        pltpu.make_async_copy(o_v[b].at[pl.ds(0, D)],
                              w128_out.at[pl.ds(0, D)], osems[b]).wait()

    @pl.when(wid == 4)
    def _():
        pltpu.async_copy(tail32, o0.at[pl.ds(0, 32)], li0).wait()
        pltpu.async_copy(o0.at[pl.ds(0, 32)],
                         w128_out.at[pl.ds(NFB * D, 32)], lo0).wait()


def kernel(input_, weight):
    w128 = weight.reshape(500000, 128)
    idx_t = input_.T
    out = _emb2(w128, idx_t)
    return out.transpose(2, 0, 1)


# speedup vs baseline: 1.7720x; 1.0493x over previous
import functools

import jax
import jax.numpy as jnp
from jax import lax
from jax.experimental import pallas as pl
from jax.experimental.pallas import tpu as pltpu
from jax.experimental.pallas import tpu_sc as plsc

D = 64
B = 16384 * 50
NC, NS = 2, 16
NW = NC * NS
B_PER_W = B // NW
CH = 512
NBUF = 2
NCHUNK = B_PER_W // CH

_mesh = plsc.VectorSubcoreMesh(core_axis_name="c", subcore_axis_name="s")


@functools.partial(
    pl.kernel,
    mesh=_mesh,
    out_type=jax.ShapeDtypeStruct((B, D), jnp.float32),
    scratch_types=[
        pltpu.VMEM((CH,), jnp.int32),
        pltpu.VMEM((CH,), jnp.int32),
        pltpu.VMEM((CH, D), jnp.float32),
        pltpu.VMEM((CH, D), jnp.float32),
        pltpu.SemaphoreType.DMA,
        pltpu.SemaphoreType.DMA,
        pltpu.SemaphoreType.DMA,
        pltpu.SemaphoreType.DMA,
        pltpu.SemaphoreType.DMA,
        pltpu.SemaphoreType.DMA,
    ],
    compiler_params=pltpu.CompilerParams(use_tc_tiling_on_sc=False),
)
def _emb_lookup(table_hbm, idx_hbm, out_hbm,
                idx_v0, idx_v1, rows_v0, rows_v1,
                isem0, isem1, gsem0, gsem1, ssem0, ssem1):
    idx_v = (idx_v0, idx_v1)
    rows_v = (rows_v0, rows_v1)
    isems = (isem0, isem1)
    gsems = (gsem0, gsem1)
    ssems = (ssem0, ssem1)

    wid = lax.axis_index("s") * NC + lax.axis_index("c")
    base = wid * B_PER_W

    for b in range(NBUF):
        pltpu.async_copy(
            idx_hbm.at[pl.ds(base + b * CH, CH)], idx_v[b], isems[b])

    def body(i, carry):
        g0 = i * NBUF
        for b in range(NBUF):
            off = base + (g0 + b) * CH

            @pl.when(i > 0)
            def _wait_store(b=b, off=off):
                pltpu.make_async_copy(
                    rows_v[b], out_hbm.at[pl.ds(base, CH)], ssems[b]).wait()

            pltpu.make_async_copy(
                idx_hbm.at[pl.ds(off, CH)], idx_v[b], isems[b]).wait()
            pltpu.async_copy(table_hbm.at[idx_v[b]], rows_v[b], gsems[b])

        for b in range(NBUF):
            off = base + (g0 + b) * CH
            pltpu.make_async_copy(
                table_hbm.at[idx_v[b]], rows_v[b], gsems[b]).wait()

            @pl.when(g0 + b + NBUF < NCHUNK)
            def _prefetch_idx(b=b, off=off):
                pltpu.async_copy(
                    idx_hbm.at[pl.ds(off + NBUF * CH, CH)], idx_v[b], isems[b])

            pltpu.async_copy(rows_v[b], out_hbm.at[pl.ds(off, CH)], ssems[b])
        return carry

    lax.fori_loop(0, NCHUNK // NBUF, body, 0)

    for b in range(NBUF):
        pltpu.make_async_copy(
            rows_v[b], out_hbm.at[pl.ds(base, CH)], ssems[b]).wait()


def kernel(input_, weight):
    bsz, hist = input_.shape
    idx = input_.reshape(-1).astype(jnp.int32)
    out = _emb_lookup(weight, idx)
    return out.reshape(bsz, hist, D)
